# Initial kernel scaffold; baseline (speedup 1.0000x reference)
import jax
import jax.numpy as jnp
from jax.experimental import pallas as pl


def kernel(word_ids, modalities_ids, age_ids, delays_ids, seg_ids, posi_ids, NPI_ids, word_table, seg_table, mod_table, age_table, delays_table, posi_table, gamma, beta):
    raise NotImplementedError("write your pallas kernel here")


# speedup vs baseline: 1.0390x; 1.0390x over previous
import functools

import jax
import jax.numpy as jnp
from jax import lax
from jax.experimental import pallas as pl
from jax.experimental.pallas import tpu as pltpu
from jax.experimental.pallas import tpu_sc as plsc

B, L, H = 1024, 200, 128
TOK = B * L
NC, NS = 2, 16
NW = NC * NS
TOK_PER_W = TOK // NW
T = 64
NCHUNK = TOK_PER_W // T
EPS = 1e-12
import numpy as np

_RSQRT_MAGIC = np.int32(0x5F3759DF)


def _xlane_sum(v):
    lanes = lax.iota(jnp.int32, 16)
    for s in (8, 4, 2, 1):
        v = v + v.at[lanes ^ s].get(mode="promise_in_bounds", unique_indices=True)
    return v


def _rsqrt(x):
    i = plsc.bitcast(x, jnp.int32)
    i = _RSQRT_MAGIC - lax.shift_right_logical(i, 1)
    y = plsc.bitcast(i, jnp.float32)
    for _ in range(3):
        y = y * (1.5 - 0.5 * x * y * y)
    return y


def _sc_body(w_ids, s_ids, m_ids, a_ids, d_ids, n_ids, p_ids,
             wt, st, mt, at_, dt, ptab, gamma, beta,
             out, idx_v, bufs, obuf, gbv):
    ids = (w_ids, s_ids, m_ids, a_ids, d_ids, n_ids, p_ids)
    tabs = (wt, st, mt, at_, dt, dt, ptab)

    wid = lax.axis_index("s") * NC + lax.axis_index("c")
    base = wid * TOK_PER_W

    pltpu.sync_copy(gamma, gbv.at[0])
    pltpu.sync_copy(beta, gbv.at[1])
    g = [gbv[0, pl.ds(c * 16, 16)] for c in range(8)]
    bta = [gbv[1, pl.ds(c * 16, 16)] for c in range(8)]

    def chunk(i, carry):
        start = base + i * T
        for j in range(7):
            pltpu.sync_copy(ids[j].at[pl.ds(start, T)], idx_v.at[j])
        for j in range(7):
            pltpu.sync_copy(tabs[j].at[idx_v.at[j]], bufs.at[j])

        def tok(t, c2):
            acc = [bufs[0, t, pl.ds(c * 16, 16)] for c in range(8)]
            for j in range(1, 7):
                for c in range(8):
                    acc[c] = acc[c] + bufs[j, t, pl.ds(c * 16, 16)]
            vsum = acc[0]
            for c in range(1, 8):
                vsum = vsum + acc[c]
            mu = _xlane_sum(vsum) * (1.0 / H)
            d = [acc[c] - mu for c in range(8)]
            vsq = d[0] * d[0]
            for c in range(1, 8):
                vsq = vsq + d[c] * d[c]
            var = _xlane_sum(vsq) * (1.0 / H)
            inv = _rsqrt(var + EPS)
            for c in range(8):
                obuf[t, pl.ds(c * 16, 16)] = d[c] * inv * g[c] + bta[c]
            return c2

        lax.fori_loop(0, T, tok, 0)
        pltpu.sync_copy(obuf, out.at[pl.ds(start, T)])
        return carry

    lax.fori_loop(0, NCHUNK, chunk, 0)


@jax.jit
def _run(w_ids, s_ids, m_ids, a_ids, d_ids, n_ids, p_ids,
         wt, st, mt, at_, dt, ptab, gamma, beta):
    mesh = plsc.VectorSubcoreMesh(core_axis_name="c", subcore_axis_name="s")
    f = pl.kernel(
        _sc_body,
        out_type=jax.ShapeDtypeStruct((TOK, H), jnp.float32),
        mesh=mesh,
        scratch_types=[
            pltpu.VMEM((7, T), jnp.int32),
            pltpu.VMEM((7, T, H), jnp.float32),
            pltpu.VMEM((T, H), jnp.float32),
            pltpu.VMEM((2, H), jnp.float32),
        ],
        compiler_params=pltpu.CompilerParams(needs_layout_passes=False),
    )
    return f(w_ids, s_ids, m_ids, a_ids, d_ids, n_ids, p_ids,
             wt, st, mt, at_, dt, ptab, gamma, beta)


def kernel(word_ids, modalities_ids, age_ids, delays_ids, seg_ids, posi_ids,
           NPI_ids, word_table, seg_table, mod_table, age_table, delays_table,
           posi_table, gamma, beta):
    flat = lambda x: x.reshape(-1).astype(jnp.int32)
    out = _run(flat(word_ids), flat(seg_ids), flat(modalities_ids),
               flat(age_ids), flat(delays_ids), flat(NPI_ids), flat(posi_ids),
               word_table, seg_table, mod_table, age_table, delays_table,
               posi_table, gamma, beta)
    return out.reshape(B, L, H)


# speedup vs baseline: 3.8030x; 3.6603x over previous
import functools

import jax
import jax.numpy as jnp
import numpy as np
from jax import lax
from jax.experimental import pallas as pl
from jax.experimental.pallas import tpu as pltpu
from jax.experimental.pallas import tpu_sc as plsc

B, L, H = 1024, 200, 128
TOK = B * L
NC, NS = 2, 16
NW = NC * NS
TOK_PER_W = TOK // NW
T = 64
NCHUNK = TOK_PER_W // T
EPS = 1e-12
_RSQRT_MAGIC = np.int32(0x5F3759DF)


def _xlane_sum(v):
    lanes = lax.iota(jnp.int32, 16)
    for s in (8, 4, 2, 1):
        v = v + v.at[lanes ^ s].get(mode="promise_in_bounds", unique_indices=True)
    return v


def _rsqrt(x):
    i = plsc.bitcast(x, jnp.int32)
    i = _RSQRT_MAGIC - lax.shift_right_logical(i, 1)
    y = plsc.bitcast(i, jnp.float32)
    for _ in range(3):
        y = y * (1.5 - 0.5 * x * y * y)
    return y


def _sc_body(w_ids, s_ids, m_ids, a_ids, d_ids, n_ids, p_ids,
             wt, st, mt, at_, dt, ptab, gamma, beta,
             out, idx_v, wbuf, pbuf, obuf, gbv,
             seg_v, mod_v, age_v, del_v):
    ids = (w_ids, s_ids, m_ids, a_ids, d_ids, n_ids, p_ids)
    wid = lax.axis_index("s") * NC + lax.axis_index("c")
    base = wid * TOK_PER_W

    pltpu.sync_copy(st, seg_v)
    pltpu.sync_copy(mt, mod_v)
    pltpu.sync_copy(at_, age_v)
    pltpu.sync_copy(dt, del_v)
    pltpu.sync_copy(gamma, gbv.at[0])
    pltpu.sync_copy(beta, gbv.at[1])
    g = [gbv[0, pl.ds(c * 16, 16)] for c in range(8)]
    bta = [gbv[1, pl.ds(c * 16, 16)] for c in range(8)]
    cols = [lax.iota(jnp.int32, 16) + c * 16 for c in range(8)]
    lanes = lax.iota(jnp.int32, 16)
    small = ((seg_v, 1), (mod_v, 2), (age_v, 3), (del_v, 4), (del_v, 5))

    def chunk(i, carry):
        start = base + i * T
        for j in range(7):
            pltpu.sync_copy(ids[j].at[pl.ds(start, T)], idx_v.at[j])
        pltpu.sync_copy(wt.at[idx_v.at[0]], wbuf)
        pltpu.sync_copy(ptab.at[idx_v.at[6]], pbuf)

        def tok(t, c2):
            t16 = (t // 16) * 16
            lane = jnp.full((16,), t - t16, jnp.int32)
            acc = [wbuf[t, pl.ds(c * 16, 16)] + pbuf[t, pl.ds(c * 16, 16)]
                   for c in range(8)]
            for tab, j in small:
                idvec = idx_v[j, pl.ds(t16, 16)]
                row = idvec.at[lane].get(mode="promise_in_bounds")
                for c in range(8):
                    acc[c] = acc[c] + plsc.load_gather(tab, [row, cols[c]])
            vsum = acc[0]
            for c in range(1, 8):
                vsum = vsum + acc[c]
            mu = _xlane_sum(vsum) * (1.0 / H)
            d = [acc[c] - mu for c in range(8)]
            vsq = d[0] * d[0]
            for c in range(1, 8):
                vsq = vsq + d[c] * d[c]
            var = _xlane_sum(vsq) * (1.0 / H)
            inv = _rsqrt(var + EPS)
            for c in range(8):
                obuf[t, pl.ds(c * 16, 16)] = d[c] * inv * g[c] + bta[c]
            return c2

        lax.fori_loop(0, T, tok, 0, unroll=2)
        pltpu.sync_copy(obuf, out.at[pl.ds(start, T)])
        return carry

    lax.fori_loop(0, NCHUNK, chunk, 0)


@jax.jit
def _run(w_ids, s_ids, m_ids, a_ids, d_ids, n_ids, p_ids,
         wt, st, mt, at_, dt, ptab, gamma, beta):
    mesh = plsc.VectorSubcoreMesh(core_axis_name="c", subcore_axis_name="s")
    f = pl.kernel(
        _sc_body,
        out_type=jax.ShapeDtypeStruct((TOK, H), jnp.float32),
        mesh=mesh,
        scratch_types=[
            pltpu.VMEM((7, T), jnp.int32),
            pltpu.VMEM((T, H), jnp.float32),
            pltpu.VMEM((T, H), jnp.float32),
            pltpu.VMEM((T, H), jnp.float32),
            pltpu.VMEM((2, H), jnp.float32),
            pltpu.VMEM((2, H), jnp.float32),
            pltpu.VMEM((10, H), jnp.float32),
            pltpu.VMEM((120, H), jnp.float32),
            pltpu.VMEM((365, H), jnp.float32),
        ],
        compiler_params=pltpu.CompilerParams(needs_layout_passes=False),
    )
    return f(w_ids, s_ids, m_ids, a_ids, d_ids, n_ids, p_ids,
             wt, st, mt, at_, dt, ptab, gamma, beta)


def kernel(word_ids, modalities_ids, age_ids, delays_ids, seg_ids, posi_ids,
           NPI_ids, word_table, seg_table, mod_table, age_table, delays_table,
           posi_table, gamma, beta):
    flat = lambda x: x.reshape(-1).astype(jnp.int32)
    out = _run(flat(word_ids), flat(seg_ids), flat(modalities_ids),
               flat(age_ids), flat(delays_ids), flat(NPI_ids), flat(posi_ids),
               word_table, seg_table, mod_table, age_table, delays_table,
               posi_table, gamma, beta)
    return out.reshape(B, L, H)


# speedup vs baseline: 5.5184x; 1.4511x over previous
import functools

import jax
import jax.numpy as jnp
import numpy as np
from jax import lax
from jax.experimental import pallas as pl
from jax.experimental.pallas import tpu as pltpu
from jax.experimental.pallas import tpu_sc as plsc

B, L, H = 1024, 200, 128
TOK = B * L
NC, NS = 2, 16
NW = NC * NS
TOK_PER_W = TOK // NW
T = 64
NCHUNK = TOK_PER_W // T
EPS = 1e-12
_RSQRT_MAGIC = np.int32(0x5F3759DF)


def _xlane_sum(v):
    lanes = lax.iota(jnp.int32, 16)
    for s in (8, 4, 2, 1):
        v = v + v.at[lanes ^ s].get(mode="promise_in_bounds", unique_indices=True)
    return v


def _rsqrt(x):
    i = plsc.bitcast(x, jnp.int32)
    i = _RSQRT_MAGIC - lax.shift_right_logical(i, 1)
    y = plsc.bitcast(i, jnp.float32)
    for _ in range(3):
        y = y * (1.5 - 0.5 * x * y * y)
    return y


def _sc_body(ids_all, wt, st, mt, at_, dt, ptab, gamma, beta,
             out, idx_pf, idx_cur, wbuf, pbuf, obuf, gbv,
             seg_v, mod_v, age_v, del_v,
             sem_pf, sem_cur, sem_g, sem_out):
    wid = lax.axis_index("s") * NC + lax.axis_index("c")
    base = wid * TOK_PER_W

    pltpu.sync_copy(st, seg_v)
    pltpu.sync_copy(mt, mod_v)
    pltpu.sync_copy(at_, age_v)
    pltpu.sync_copy(dt, del_v)
    pltpu.sync_copy(gamma, gbv.at[0])
    pltpu.sync_copy(beta, gbv.at[1])
    g = [gbv[0, pl.ds(c * 16, 16)] for c in range(8)]
    bta = [gbv[1, pl.ds(c * 16, 16)] for c in range(8)]
    cols = [lax.iota(jnp.int32, 16) + c * 16 for c in range(8)]
    small = (seg_v, mod_v, age_v, del_v, del_v)

    def pf_issue(i, q):
        pltpu.async_copy(ids_all.at[wid, i], idx_pf.at[q], sem_pf)

    def pf_wait():
        pltpu.make_async_copy(ids_all.at[wid, 0], idx_pf.at[0], sem_pf).wait()

    def cur_issue(i):
        pltpu.async_copy(ids_all.at[wid, i], idx_cur, sem_cur)

    def cur_wait():
        pltpu.make_async_copy(ids_all.at[wid, 0], idx_cur, sem_cur).wait()

    def gather_issue(q, p):
        pltpu.async_copy(wt.at[idx_pf.at[q, 0, pl.ds(0, T)]], wbuf.at[p], sem_g)
        pltpu.async_copy(ptab.at[idx_pf.at[q, 6, pl.ds(0, T)]], pbuf.at[p], sem_g)

    def gather_wait():
        pltpu.make_async_copy(
            wt.at[idx_pf.at[0, 0, pl.ds(0, T)]], wbuf.at[0], sem_g).wait()
        pltpu.make_async_copy(
            ptab.at[idx_pf.at[0, 6, pl.ds(0, T)]], pbuf.at[0], sem_g).wait()

    def out_issue(i, p):
        pltpu.async_copy(obuf.at[p], out.at[pl.ds(base + i * T, T)], sem_out)

    def out_wait():
        pltpu.make_async_copy(obuf.at[0], out.at[pl.ds(base, T)], sem_out).wait()

    cur_issue(0)
    pf_issue(0, 0)
    pf_wait()
    gather_issue(0, 0)
    pf_issue(1, 1)

    def chunk(i, carry):
        p = lax.rem(i, 2)
        q1 = lax.rem(i + 1, 2)
        gather_wait()

        @pl.when(i > 0)
        def _free_outbuf():
            out_wait()

        cur_wait()
        rv = [[idx_cur[1 + j, pl.ds(gg * 16, 16)] for gg in range(4)]
              for j in range(5)]

        @pl.when(i + 1 < NCHUNK)
        def _next_cur():
            cur_issue(i + 1)

        @pl.when(i + 1 < NCHUNK)
        def _next_gathers():
            pf_wait()
            gather_issue(q1, 1 - p)

        @pl.when(i + 2 < NCHUNK)
        def _next_pf():
            pf_issue(i + 2, lax.rem(i, 2))

        for gg in range(4):
            def tok(t, c2, gg=gg):
                lane = jnp.full((16,), t, jnp.int32)
                tt = gg * 16 + t
                acc = [wbuf[p, tt, pl.ds(c * 16, 16)]
                       + pbuf[p, tt, pl.ds(c * 16, 16)] for c in range(8)]
                for j in range(5):
                    row = rv[j][gg].at[lane].get(mode="promise_in_bounds")
                    for c in range(8):
                        acc[c] = acc[c] + plsc.load_gather(small[j], [row, cols[c]])
                vsum = acc[0]
                for c in range(1, 8):
                    vsum = vsum + acc[c]
                mu = _xlane_sum(vsum) * (1.0 / H)
                d = [acc[c] - mu for c in range(8)]
                vsq = d[0] * d[0]
                for c in range(1, 8):
                    vsq = vsq + d[c] * d[c]
                var = _xlane_sum(vsq) * (1.0 / H)
                inv = _rsqrt(var + EPS)
                for c in range(8):
                    obuf[p, tt, pl.ds(c * 16, 16)] = d[c] * inv * g[c] + bta[c]
                return c2

            lax.fori_loop(0, 16, tok, 0, unroll=2)

        out_issue(i, p)
        return carry

    lax.fori_loop(0, NCHUNK, chunk, 0)
    out_wait()


@jax.jit
def _run(ids_all, wt, st, mt, at_, dt, ptab, gamma, beta):
    mesh = plsc.VectorSubcoreMesh(core_axis_name="c", subcore_axis_name="s")
    f = pl.kernel(
        _sc_body,
        out_type=jax.ShapeDtypeStruct((TOK, H), jnp.float32),
        mesh=mesh,
        scratch_types=[
            pltpu.VMEM((2, 7, 128), jnp.int32),
            pltpu.VMEM((7, 128), jnp.int32),
            pltpu.VMEM((2, T, H), jnp.float32),
            pltpu.VMEM((2, T, H), jnp.float32),
            pltpu.VMEM((2, T, H), jnp.float32),
            pltpu.VMEM((2, H), jnp.float32),
            pltpu.VMEM((2, H), jnp.float32),
            pltpu.VMEM((10, H), jnp.float32),
            pltpu.VMEM((120, H), jnp.float32),
            pltpu.VMEM((365, H), jnp.float32),
            pltpu.SemaphoreType.DMA,
            pltpu.SemaphoreType.DMA,
            pltpu.SemaphoreType.DMA,
            pltpu.SemaphoreType.DMA,
        ],
        compiler_params=pltpu.CompilerParams(needs_layout_passes=False),
    )
    return f(ids_all, wt, st, mt, at_, dt, ptab, gamma, beta)


def kernel(word_ids, modalities_ids, age_ids, delays_ids, seg_ids, posi_ids,
           NPI_ids, word_table, seg_table, mod_table, age_table, delays_table,
           posi_table, gamma, beta):
    flat = lambda x: x.reshape(-1).astype(jnp.int32)
    ids_all = jnp.stack([
        flat(word_ids), flat(seg_ids), flat(modalities_ids), flat(age_ids),
        flat(delays_ids), flat(NPI_ids), flat(posi_ids)])
    ids_all = ids_all.reshape(7, NW, NCHUNK, T).transpose(1, 2, 0, 3)
    ids_all = jnp.pad(ids_all, ((0, 0), (0, 0), (0, 0), (0, 128 - T)))
    out = _run(ids_all, word_table, seg_table, mod_table, age_table,
               delays_table, posi_table, gamma, beta)
    return out.reshape(B, L, H)


# speedup vs baseline: 6.1493x; 1.1143x over previous
import functools

import jax
import jax.numpy as jnp
import numpy as np
from jax import lax
from jax.experimental import pallas as pl
from jax.experimental.pallas import tpu as pltpu
from jax.experimental.pallas import tpu_sc as plsc

B, L, H = 1024, 200, 128
TOK = B * L
NC, NS = 2, 16
NW = NC * NS
TOK_PER_W = TOK // NW
T = 64
NCHUNK = TOK_PER_W // T
EPS = 1e-12
_RSQRT_MAGIC = np.int32(0x5F3759DF)


def _xlane_sum(v):
    lanes = lax.iota(jnp.int32, 16)
    for s in (8, 4, 2, 1):
        v = v + v.at[lanes ^ s].get(mode="promise_in_bounds", unique_indices=True)
    return v


def _rsqrt(x):
    i = plsc.bitcast(x, jnp.int32)
    i = _RSQRT_MAGIC - lax.shift_right_logical(i, 1)
    y = plsc.bitcast(i, jnp.float32)
    for _ in range(3):
        y = y * (1.5 - 0.5 * x * y * y)
    return y


def _sc_body(ids_all, wt, st, mt, at_, dt, ptab, gamma, beta,
             out, idx_pf, idx_cur, wbuf, pbuf, gbv,
             seg_v, mod_v, age_v, del_v,
             sem_pf, sem_cur, sem_g, sem_out):
    wid = lax.axis_index("s") * NC + lax.axis_index("c")
    base = wid * TOK_PER_W

    pltpu.sync_copy(st, seg_v)
    pltpu.sync_copy(mt, mod_v)
    pltpu.sync_copy(at_, age_v)
    pltpu.sync_copy(dt, del_v)
    pltpu.sync_copy(gamma, gbv.at[0])
    pltpu.sync_copy(beta, gbv.at[1])
    g = [gbv[0, pl.ds(c * 16, 16)] for c in range(8)]
    bta = [gbv[1, pl.ds(c * 16, 16)] for c in range(8)]
    cols = [lax.iota(jnp.int32, 16) + c * 16 for c in range(8)]
    small = (seg_v, mod_v, age_v, del_v, del_v)

    def pf_issue(i, q):
        pltpu.async_copy(ids_all.at[wid, i], idx_pf.at[q], sem_pf)

    def pf_wait():
        pltpu.make_async_copy(ids_all.at[wid, 0], idx_pf.at[0], sem_pf).wait()

    def cur_issue(i):
        pltpu.async_copy(ids_all.at[wid, i], idx_cur, sem_cur)

    def cur_wait():
        pltpu.make_async_copy(ids_all.at[wid, 0], idx_cur, sem_cur).wait()

    def gather_issue(q, p):
        pltpu.async_copy(wt.at[idx_pf.at[q, 0, pl.ds(0, T)]], wbuf.at[p], sem_g)
        pltpu.async_copy(ptab.at[idx_pf.at[q, 6, pl.ds(0, T)]], pbuf.at[p], sem_g)

    def gather_wait():
        pltpu.make_async_copy(
            wt.at[idx_pf.at[0, 0, pl.ds(0, T)]], wbuf.at[0], sem_g).wait()
        pltpu.make_async_copy(
            ptab.at[idx_pf.at[0, 6, pl.ds(0, T)]], pbuf.at[0], sem_g).wait()

    def out_issue(i, p):
        pltpu.async_copy(wbuf.at[p], out.at[pl.ds(base + i * T, T)], sem_out)

    def out_wait():
        pltpu.make_async_copy(wbuf.at[0], out.at[pl.ds(base, T)], sem_out).wait()

    cur_issue(0)
    pf_issue(0, 0)
    pf_wait()
    gather_issue(0, 0)
    pf_issue(1, 1)

    def chunk(i, carry):
        p = lax.rem(i, 2)
        q1 = lax.rem(i + 1, 2)
        gather_wait()

        @pl.when(i > 0)
        def _free_outbuf():
            out_wait()

        cur_wait()
        rv = [[idx_cur[1 + j, pl.ds(gg * 16, 16)] for gg in range(4)]
              for j in range(5)]

        @pl.when(i + 1 < NCHUNK)
        def _next_cur():
            cur_issue(i + 1)

        @pl.when(i + 1 < NCHUNK)
        def _next_gathers():
            pf_wait()
            gather_issue(q1, 1 - p)

        @pl.when(i + 2 < NCHUNK)
        def _next_pf():
            pf_issue(i + 2, lax.rem(i, 2))

        for gg in range(4):
            def tok(t, c2, gg=gg):
                lane = jnp.full((16,), t, jnp.int32)
                tt = gg * 16 + t
                acc = [wbuf[p, tt, pl.ds(c * 16, 16)]
                       + pbuf[p, tt, pl.ds(c * 16, 16)] for c in range(8)]
                for j in range(5):
                    row = rv[j][gg].at[lane].get(mode="promise_in_bounds")
                    for c in range(8):
                        acc[c] = acc[c] + plsc.load_gather(small[j], [row, cols[c]])
                sq = [acc[c] * acc[c] for c in range(8)]
                vsum, vsq = acc[0], sq[0]
                for c in range(1, 8):
                    vsum = vsum + acc[c]
                    vsq = vsq + sq[c]
                mu = _xlane_sum(vsum) * (1.0 / H)
                var = _xlane_sum(vsq) * (1.0 / H) - mu * mu
                inv = _rsqrt(var + EPS)
                for c in range(8):
                    k = inv * g[c]
                    wbuf[p, tt, pl.ds(c * 16, 16)] = acc[c] * k + (bta[c] - mu * k)
                return c2

            lax.fori_loop(0, 16, tok, 0, unroll=4)

        out_issue(i, p)
        return carry

    lax.fori_loop(0, NCHUNK, chunk, 0)
    out_wait()


@jax.jit
def _run(ids_all, wt, st, mt, at_, dt, ptab, gamma, beta):
    mesh = plsc.VectorSubcoreMesh(core_axis_name="c", subcore_axis_name="s")
    f = pl.kernel(
        _sc_body,
        out_type=jax.ShapeDtypeStruct((TOK, H), jnp.float32),
        mesh=mesh,
        scratch_types=[
            pltpu.VMEM((2, 7, 128), jnp.int32),
            pltpu.VMEM((7, 128), jnp.int32),
            pltpu.VMEM((2, T, H), jnp.float32),
            pltpu.VMEM((2, T, H), jnp.float32),
            pltpu.VMEM((2, H), jnp.float32),
            pltpu.VMEM((2, H), jnp.float32),
            pltpu.VMEM((10, H), jnp.float32),
            pltpu.VMEM((120, H), jnp.float32),
            pltpu.VMEM((365, H), jnp.float32),
            pltpu.SemaphoreType.DMA,
            pltpu.SemaphoreType.DMA,
            pltpu.SemaphoreType.DMA,
            pltpu.SemaphoreType.DMA,
        ],
        compiler_params=pltpu.CompilerParams(needs_layout_passes=False),
    )
    return f(ids_all, wt, st, mt, at_, dt, ptab, gamma, beta)


def kernel(word_ids, modalities_ids, age_ids, delays_ids, seg_ids, posi_ids,
           NPI_ids, word_table, seg_table, mod_table, age_table, delays_table,
           posi_table, gamma, beta):
    flat = lambda x: x.reshape(-1).astype(jnp.int32)
    ids_all = jnp.stack([
        flat(word_ids), flat(seg_ids), flat(modalities_ids), flat(age_ids),
        flat(delays_ids), flat(NPI_ids), flat(posi_ids)])
    ids_all = ids_all.reshape(7, NW, NCHUNK, T).transpose(1, 2, 0, 3)
    ids_all = jnp.pad(ids_all, ((0, 0), (0, 0), (0, 0), (0, 128 - T)))
    out = _run(ids_all, word_table, seg_table, mod_table, age_table,
               delays_table, posi_table, gamma, beta)
    return out.reshape(B, L, H)


# speedup vs baseline: 6.8672x; 1.1167x over previous
import functools

import jax
import jax.numpy as jnp
import numpy as np
from jax import lax
from jax.experimental import pallas as pl
from jax.experimental.pallas import tpu as pltpu
from jax.experimental.pallas import tpu_sc as plsc

B, L, H = 1024, 200, 128
TOK = B * L
NC, NS = 2, 16
NW = NC * NS
TOK_PER_W = TOK // NW
T = 64
NCHUNK = TOK_PER_W // T
EPS = 1e-12
_RSQRT_MAGIC = np.int32(0x5F3759DF)


def _xlane_sum(v):
    lanes = lax.iota(jnp.int32, 16)
    for s in (8, 4, 2, 1):
        v = v + v.at[lanes ^ s].get(mode="promise_in_bounds", unique_indices=True)
    return v


def _rsqrt(x):
    i = plsc.bitcast(x, jnp.int32)
    i = _RSQRT_MAGIC - lax.shift_right_logical(i, 1)
    y = plsc.bitcast(i, jnp.float32)
    for _ in range(2):
        y = y * (1.5 - 0.5 * x * y * y)
    return y


def _sc_body(ids_all, wt, st, mt, at_, dt, ptab, gamma, beta,
             out, idx_pf, idx_cur, wbuf, pbuf, gbv,
             seg_v, mod_v, sm_v, age_v, del_v,
             sem_pf, sem_cur, sem_g, sem_out):
    wid = lax.axis_index("s") * NC + lax.axis_index("c")
    base = wid * TOK_PER_W

    pltpu.sync_copy(st, seg_v)
    pltpu.sync_copy(mt, mod_v)
    pltpu.sync_copy(at_, age_v)
    pltpu.sync_copy(dt, del_v)
    pltpu.sync_copy(gamma, gbv.at[0])
    pltpu.sync_copy(beta, gbv.at[1])
    g = [gbv[0, pl.ds(c * 16, 16)] for c in range(8)]
    bta = [gbv[1, pl.ds(c * 16, 16)] for c in range(8)]
    cols = [lax.iota(jnp.int32, 16) + c * 16 for c in range(8)]
    for s2 in range(2):
        for m in range(10):
            for c in range(8):
                sm_v[s2 * 10 + m, pl.ds(c * 16, 16)] = (
                    seg_v[s2, pl.ds(c * 16, 16)] + mod_v[m, pl.ds(c * 16, 16)])
    small = (sm_v, age_v, del_v, del_v)

    def pf_issue(i, q):
        pltpu.async_copy(ids_all.at[wid, i], idx_pf.at[q], sem_pf)

    def pf_wait():
        pltpu.make_async_copy(ids_all.at[wid, 0], idx_pf.at[0], sem_pf).wait()

    def cur_issue(i):
        pltpu.async_copy(ids_all.at[wid, i], idx_cur, sem_cur)

    def cur_wait():
        pltpu.make_async_copy(ids_all.at[wid, 0], idx_cur, sem_cur).wait()

    def gather_issue(q, p):
        pltpu.async_copy(wt.at[idx_pf.at[q, 0, pl.ds(0, T)]], wbuf.at[p], sem_g)
        pltpu.async_copy(ptab.at[idx_pf.at[q, 6, pl.ds(0, T)]], pbuf.at[p], sem_g)

    def gather_wait():
        pltpu.make_async_copy(
            wt.at[idx_pf.at[0, 0, pl.ds(0, T)]], wbuf.at[0], sem_g).wait()
        pltpu.make_async_copy(
            ptab.at[idx_pf.at[0, 6, pl.ds(0, T)]], pbuf.at[0], sem_g).wait()

    def out_issue(i, p):
        pltpu.async_copy(wbuf.at[p], out.at[pl.ds(base + i * T, T)], sem_out)

    def out_wait():
        pltpu.make_async_copy(wbuf.at[0], out.at[pl.ds(base, T)], sem_out).wait()

    cur_issue(0)
    pf_issue(0, 0)
    pf_wait()
    gather_issue(0, 0)
    pf_issue(1, 1)

    def chunk(i, carry):
        p = lax.rem(i, 2)
        q1 = lax.rem(i + 1, 2)
        gather_wait()

        @pl.when(i > 0)
        def _free_outbuf():
            out_wait()

        cur_wait()
        rv5 = [[idx_cur[1 + j, pl.ds(gg * 16, 16)] for gg in range(4)]
               for j in range(5)]
        rv = [[rv5[0][gg] * 10 + rv5[1][gg] for gg in range(4)],
              rv5[2], rv5[3], rv5[4]]

        @pl.when(i + 1 < NCHUNK)
        def _next_cur():
            cur_issue(i + 1)

        @pl.when(i + 1 < NCHUNK)
        def _next_gathers():
            pf_wait()
            gather_issue(q1, 1 - p)

        @pl.when(i + 2 < NCHUNK)
        def _next_pf():
            pf_issue(i + 2, lax.rem(i, 2))

        for gg in range(4):
            def tok(t, c2, gg=gg):
                lane = jnp.full((16,), t, jnp.int32)
                tt = gg * 16 + t
                acc = [wbuf[p, tt, pl.ds(c * 16, 16)]
                       + pbuf[p, tt, pl.ds(c * 16, 16)] for c in range(8)]
                for j in range(4):
                    row = rv[j][gg].at[lane].get(mode="promise_in_bounds")
                    for c in range(8):
                        acc[c] = acc[c] + plsc.load_gather(small[j], [row, cols[c]])
                sq = [acc[c] * acc[c] for c in range(8)]
                s4 = [acc[2 * c] + acc[2 * c + 1] for c in range(4)]
                q4 = [sq[2 * c] + sq[2 * c + 1] for c in range(4)]
                vsum = (s4[0] + s4[1]) + (s4[2] + s4[3])
                vsq = (q4[0] + q4[1]) + (q4[2] + q4[3])
                mu = _xlane_sum(vsum) * (1.0 / H)
                var = _xlane_sum(vsq) * (1.0 / H) - mu * mu
                inv = _rsqrt(var + EPS)
                for c in range(8):
                    k = inv * g[c]
                    wbuf[p, tt, pl.ds(c * 16, 16)] = acc[c] * k + (bta[c] - mu * k)
                return c2

            lax.fori_loop(0, 16, tok, 0, unroll=4)

        out_issue(i, p)
        return carry

    lax.fori_loop(0, NCHUNK, chunk, 0)
    out_wait()


@jax.jit
def _run(ids_all, wt, st, mt, at_, dt, ptab, gamma, beta):
    mesh = plsc.VectorSubcoreMesh(core_axis_name="c", subcore_axis_name="s")
    f = pl.kernel(
        _sc_body,
        out_type=jax.ShapeDtypeStruct((TOK, H), jnp.float32),
        mesh=mesh,
        scratch_types=[
            pltpu.VMEM((2, 7, 128), jnp.int32),
            pltpu.VMEM((7, 128), jnp.int32),
            pltpu.VMEM((2, T, H), jnp.float32),
            pltpu.VMEM((2, T, H), jnp.float32),
            pltpu.VMEM((2, H), jnp.float32),
            pltpu.VMEM((2, H), jnp.float32),
            pltpu.VMEM((10, H), jnp.float32),
            pltpu.VMEM((20, H), jnp.float32),
            pltpu.VMEM((120, H), jnp.float32),
            pltpu.VMEM((365, H), jnp.float32),
            pltpu.SemaphoreType.DMA,
            pltpu.SemaphoreType.DMA,
            pltpu.SemaphoreType.DMA,
            pltpu.SemaphoreType.DMA,
        ],
        compiler_params=pltpu.CompilerParams(needs_layout_passes=False),
    )
    return f(ids_all, wt, st, mt, at_, dt, ptab, gamma, beta)


def kernel(word_ids, modalities_ids, age_ids, delays_ids, seg_ids, posi_ids,
           NPI_ids, word_table, seg_table, mod_table, age_table, delays_table,
           posi_table, gamma, beta):
    flat = lambda x: x.reshape(-1).astype(jnp.int32)
    ids_all = jnp.stack([
        flat(word_ids), flat(seg_ids), flat(modalities_ids), flat(age_ids),
        flat(delays_ids), flat(NPI_ids), flat(posi_ids)])
    ids_all = ids_all.reshape(7, NW, NCHUNK, T).transpose(1, 2, 0, 3)
    ids_all = jnp.pad(ids_all, ((0, 0), (0, 0), (0, 0), (0, 128 - T)))
    out = _run(ids_all, word_table, seg_table, mod_table, age_table,
               delays_table, posi_table, gamma, beta)
    return out.reshape(B, L, H)
